# baseline (device time: 104922 ns/iter reference)
import jax
import jax.numpy as jnp
from jax import lax
from jax.experimental import pallas as pl
from jax.experimental.pallas import tpu as pltpu

N_DEV = 8
BLK = 64


def kernel(x, Wq, K_ext, V_ext, Wo):
    B, sq, dm = x.shape
    _, skv, hq, dh = K_ext.shape
    dq = Wq.shape[1]

    def body(x_ref, wq_ref, k_ref, v_ref, wo_ref, out_ref,
             kfull, vfull, ksend, krecv, vsend, vrecv):
        my = lax.axis_index("i")
        left = lax.rem(my - 1 + N_DEV, N_DEV)
        right = lax.rem(my + 1, N_DEV)

        barrier = pltpu.get_barrier_semaphore()
        for nbr in [left, right]:
            pl.semaphore_signal(barrier, inc=1, device_id=(nbr,),
                                device_id_type=pl.DeviceIdType.MESH)
        pl.semaphore_wait(barrier, 2)

        kfull[my] = k_ref[...]
        vfull[my] = v_ref[...]

        for h in range(N_DEV - 1):
            c = lax.rem(my - h + N_DEV, N_DEV)
            rk = pltpu.make_async_remote_copy(
                src_ref=kfull.at[c], dst_ref=kfull.at[c],
                send_sem=ksend.at[h], recv_sem=krecv.at[h],
                device_id=(right,), device_id_type=pl.DeviceIdType.MESH)
            rv = pltpu.make_async_remote_copy(
                src_ref=vfull.at[c], dst_ref=vfull.at[c],
                send_sem=vsend.at[h], recv_sem=vrecv.at[h],
                device_id=(right,), device_id_type=pl.DeviceIdType.MESH)
            rk.start()
            rv.start()
            rk.wait()
            rv.wait()

        xv = x_ref[...]
        wq = wq_ref[...]
        wo = wo_ref[...]

        iq = lax.broadcasted_iota(jnp.int32, (sq, skv), 0)
        jk = lax.broadcasted_iota(jnp.int32, (sq, skv), 1)
        qb = (my * sq + iq) // BLK

        ks = [kfull[c].reshape(B, skv, hq * dh) for c in range(N_DEV)]
        vs = [vfull[c].reshape(B, skv, hq * dh) for c in range(N_DEV)]
        masks = [(c * skv + jk) // BLK <= qb for c in range(N_DEV)]

        dims_nt = (((1,), (1,)), ((), ()))
        dims_nn = (((1,), (0,)), ((), ()))

        for b in range(B):
            q_all = lax.dot_general(xv[b], wq, dims_nn,
                                    preferred_element_type=jnp.float32)
            ctx_heads = []
            for hh in range(hq):
                q = q_all[:, hh * dh:(hh + 1) * dh]
                s_chunks = []
                for c in range(N_DEV):
                    kc = ks[c][b][:, hh * dh:(hh + 1) * dh]
                    s = lax.dot_general(q, kc, dims_nt,
                                        preferred_element_type=jnp.float32)
                    s = s * 0.125
                    s_chunks.append(jnp.where(masks[c], s, -1e9))
                scores = jnp.concatenate(s_chunks, axis=1)
                m = jnp.max(scores, axis=1, keepdims=True)
                w = jnp.exp(scores - m)
                p = w / jnp.sum(w, axis=1, keepdims=True)
                ctx = jnp.zeros((sq, dh), jnp.float32)
                for c in range(N_DEV):
                    vc = vs[c][b][:, hh * dh:(hh + 1) * dh]
                    ctx = ctx + lax.dot_general(
                        p[:, c * skv:(c + 1) * skv], vc, dims_nn,
                        preferred_element_type=jnp.float32)
                ctx_heads.append(ctx)
            ctx_b = jnp.concatenate(ctx_heads, axis=1)
            out_ref[b] = lax.dot_general(ctx_b, wo, dims_nn,
                                         preferred_element_type=jnp.float32)

    params_cls = getattr(pltpu, "CompilerParams", None) or pltpu.TPUCompilerParams
    return pl.pallas_call(
        body,
        out_shape=jax.ShapeDtypeStruct((B, sq, dm), jnp.float32),
        in_specs=[pl.BlockSpec(memory_space=pltpu.VMEM)] * 5,
        out_specs=pl.BlockSpec(memory_space=pltpu.VMEM),
        scratch_shapes=[
            pltpu.VMEM((N_DEV, B, skv, hq, dh), jnp.float32),
            pltpu.VMEM((N_DEV, B, skv, hq, dh), jnp.float32),
            pltpu.SemaphoreType.DMA((N_DEV - 1,)),
            pltpu.SemaphoreType.DMA((N_DEV - 1,)),
            pltpu.SemaphoreType.DMA((N_DEV - 1,)),
            pltpu.SemaphoreType.DMA((N_DEV - 1,)),
        ],
        compiler_params=params_cls(collective_id=0),
    )(x, Wq, K_ext, V_ext, Wo)


# device time: 36157 ns/iter; 2.9018x vs baseline; 2.9018x over previous
import jax
import jax.numpy as jnp
from jax import lax
from jax.experimental import pallas as pl
from jax.experimental.pallas import tpu as pltpu

N_DEV = 8
BLK = 64


def kernel(x, Wq, K_ext, V_ext, Wo):
    B, sq, dm = x.shape
    _, skv, hq, dh = K_ext.shape
    dmsg = dh + 2

    def body(x_ref, wq_ref, k_ref, v_ref, wo_ref, out_ref,
             qbuf, psbuf, prbuf, qsend, qrecv, psend, precv):
        my = lax.axis_index("i")

        barrier = pltpu.get_barrier_semaphore()
        for o in range(1, N_DEV):
            pl.semaphore_signal(barrier, inc=1,
                                device_id=(lax.rem(my + o, N_DEV),),
                                device_id_type=pl.DeviceIdType.MESH)
        pl.semaphore_wait(barrier, N_DEV - 1)

        xv = x_ref[...]
        wq = wq_ref[...]
        wo = wo_ref[...]

        dims_nt = (((1,), (1,)), ((), ()))
        dims_nn = (((1,), (0,)), ((), ()))

        for b in range(B):
            q_all = lax.dot_general(xv[b], wq, dims_nn,
                                    preferred_element_type=jnp.float32)
            qbuf[my, b] = q_all.astype(jnp.bfloat16)

        for o in range(1, N_DEV):
            @pl.when(my - o >= 0)
            def _():
                pltpu.make_async_remote_copy(
                    src_ref=qbuf.at[my], dst_ref=qbuf.at[my],
                    send_sem=qsend.at[o - 1], recv_sem=qrecv.at[my],
                    device_id=(my - o,), device_id_type=pl.DeviceIdType.MESH,
                ).start()

        kl = k_ref[...].astype(jnp.bfloat16).reshape(B, skv, hq * dh)
        vl = v_ref[...].astype(jnp.bfloat16).reshape(B, skv, hq * dh)

        def partial_tiles(q_slot, mask):
            out = []
            for b in range(B):
                per_h = []
                for h in range(hq):
                    qs = q_slot[b][:, h * dh:(h + 1) * dh]
                    s = lax.dot_general(qs, kl[b][:, h * dh:(h + 1) * dh],
                                        dims_nt,
                                        preferred_element_type=jnp.float32)
                    s = s * 0.125
                    if mask is not None:
                        s = jnp.where(mask, s, -1e9)
                    m = jnp.max(s, axis=1, keepdims=True)
                    e = jnp.exp(s - m)
                    l = jnp.sum(e, axis=1, keepdims=True)
                    ctx = lax.dot_general(e.astype(jnp.bfloat16),
                                          vl[b][:, h * dh:(h + 1) * dh],
                                          dims_nn,
                                          preferred_element_type=jnp.float32)
                    per_h.append((ctx, m, l))
                out.append(per_h)
            return out

        for o in range(1, N_DEV):
            @pl.when(my + o < N_DEV)
            def _():
                s_id = my + o
                pltpu.make_async_remote_copy(
                    src_ref=qbuf.at[s_id], dst_ref=qbuf.at[s_id],
                    send_sem=qsend.at[0], recv_sem=qrecv.at[s_id],
                    device_id=(my,), device_id_type=pl.DeviceIdType.MESH,
                ).wait_recv()
                q_slot = [qbuf[s_id, b] for b in range(B)]
                tiles = partial_tiles(q_slot, None)
                for b in range(B):
                    for h in range(hq):
                        ctx, m, l = tiles[b][h]
                        msg = jnp.concatenate(
                            [ctx.astype(jnp.bfloat16),
                             m.astype(jnp.bfloat16),
                             l.astype(jnp.bfloat16)], axis=1)
                        psbuf[s_id, b, h] = msg
                pltpu.make_async_remote_copy(
                    src_ref=psbuf.at[s_id], dst_ref=prbuf.at[my],
                    send_sem=psend.at[o - 1], recv_sem=precv.at[my],
                    device_id=(s_id,), device_id_type=pl.DeviceIdType.MESH,
                ).start()

        iq = lax.broadcasted_iota(jnp.int32, (sq, skv), 0)
        jk = lax.broadcasted_iota(jnp.int32, (sq, skv), 1)
        own_mask = (jk // BLK) <= (iq // BLK)
        q_own = [qbuf[my, b] for b in range(B)]
        acc = partial_tiles(q_own, own_mask)
        acc = [[(c, m, l) for (c, m, l) in row] for row in acc]

        for c in reversed(range(N_DEV - 1)):
            @pl.when(c < my)
            def _():
                pltpu.make_async_remote_copy(
                    src_ref=prbuf.at[c], dst_ref=prbuf.at[c],
                    send_sem=psend.at[0], recv_sem=precv.at[c],
                    device_id=(my,), device_id_type=pl.DeviceIdType.MESH,
                ).wait_recv()
            valid = c < my
            for b in range(B):
                for h in range(hq):
                    msg = prbuf[c, b, h]
                    pctx = jnp.where(valid, msg[:, :dh], jnp.bfloat16(0)
                                     ).astype(jnp.float32)
                    m_c = jnp.where(valid, msg[:, dh:dh + 1], jnp.bfloat16(0)
                                    ).astype(jnp.float32)
                    m_c = jnp.where(valid, m_c, -1e9)
                    l_c = jnp.where(valid, msg[:, dh + 1:dh + 2],
                                    jnp.bfloat16(0)).astype(jnp.float32)
                    ctx_a, m_a, l_a = acc[b][h]
                    m_new = jnp.maximum(m_a, m_c)
                    alpha = jnp.exp(m_a - m_new)
                    beta = jnp.exp(m_c - m_new)
                    acc[b][h] = (ctx_a * alpha + pctx * beta,
                                 m_new,
                                 l_a * alpha + l_c * beta)

        for b in range(B):
            heads = [acc[b][h][0] / acc[b][h][2] for h in range(hq)]
            ctx_b = jnp.concatenate(heads, axis=1)
            out_ref[b] = lax.dot_general(ctx_b, wo, dims_nn,
                                         preferred_element_type=jnp.float32)

        for o in range(1, N_DEV):
            @pl.when(my - o >= 0)
            def _():
                pltpu.make_async_remote_copy(
                    src_ref=qbuf.at[my], dst_ref=qbuf.at[my],
                    send_sem=qsend.at[o - 1], recv_sem=qrecv.at[my],
                    device_id=(lax.rem(my - o + N_DEV, N_DEV),),
                    device_id_type=pl.DeviceIdType.MESH,
                ).wait_send()
            @pl.when(my + o < N_DEV)
            def _():
                pltpu.make_async_remote_copy(
                    src_ref=psbuf.at[lax.rem(my + o, N_DEV)],
                    dst_ref=prbuf.at[my],
                    send_sem=psend.at[o - 1], recv_sem=precv.at[my],
                    device_id=(lax.rem(my + o, N_DEV),),
                    device_id_type=pl.DeviceIdType.MESH,
                ).wait_send()


    params_cls = getattr(pltpu, "CompilerParams", None) or pltpu.TPUCompilerParams
    return pl.pallas_call(
        body,
        out_shape=jax.ShapeDtypeStruct((B, sq, dm), jnp.float32),
        in_specs=[pl.BlockSpec(memory_space=pltpu.VMEM)] * 5,
        out_specs=pl.BlockSpec(memory_space=pltpu.VMEM),
        scratch_shapes=[
            pltpu.VMEM((N_DEV, B, sq, hq * dh), jnp.bfloat16),
            pltpu.VMEM((N_DEV, B, hq, sq, dmsg), jnp.bfloat16),
            pltpu.VMEM((N_DEV, B, hq, sq, dmsg), jnp.bfloat16),
            pltpu.SemaphoreType.DMA((N_DEV - 1,)),
            pltpu.SemaphoreType.DMA((N_DEV,)),
            pltpu.SemaphoreType.DMA((N_DEV - 1,)),
            pltpu.SemaphoreType.DMA((N_DEV,)),
        ],
        compiler_params=params_cls(collective_id=0),
    )(x, Wq, K_ext, V_ext, Wo)
